# Initial kernel scaffold; baseline (speedup 1.0000x reference)
import jax
import jax.numpy as jnp
from jax.experimental import pallas as pl


def kernel(z, weights, index):
    raise NotImplementedError("write your pallas kernel here")


# speedup vs baseline: 3.2934x; 3.2934x over previous
import functools

import jax
import jax.numpy as jnp
from jax import lax
from jax.experimental import pallas as pl
from jax.experimental.pallas import tpu as pltpu
from jax.experimental.pallas import tpu_sc as plsc

NC = 2
NS = 16
L = 16
NW = NC * NS

B = 1024
CHUNK = 128
KSUB = B // CHUNK


@functools.partial(jax.jit, static_argnames=("n", "c"))
def _interp(zrs, idx_t, w_t, *, n, c):
    per_w = n // NW
    nblk = per_w // B
    mesh = plsc.VectorSubcoreMesh(core_axis_name="c", subcore_axis_name="s")

    @functools.partial(
        pl.kernel,
        out_type=jax.ShapeDtypeStruct((c, n), jnp.float32),
        mesh=mesh,
        scratch_types=[
            pltpu.VMEM((4, KSUB, CHUNK), jnp.int32),
            pltpu.VMEM((4, B), jnp.float32),
            pltpu.VMEM((4, B, 8), jnp.float32),
            pltpu.VMEM((8, B), jnp.float32),
            pltpu.SemaphoreType.DMA,
        ],
        compiler_params=pltpu.CompilerParams(
            needs_layout_passes=False, use_tc_tiling_on_sc=False),
    )
    def k(zrs_hbm, idx_hbm, w_hbm, out_hbm, idx_buf, w_buf, g_buf, o_buf, sem_g):
        wid = lax.axis_index("s") * NC + lax.axis_index("c")
        base = wid * per_w
        iota = lax.iota(jnp.int32, L)
        csplat = [jnp.full((L,), cc, jnp.int32) for cc in range(8)]

        @pl.loop(0, nblk)
        def _blk(s):
            qoff = pl.multiple_of(base + s * B, B)
            coff = pl.multiple_of(qoff // CHUNK, KSUB)
            pltpu.sync_copy(idx_hbm.at[:, pl.ds(coff, KSUB), :], idx_buf)
            pltpu.sync_copy(w_hbm.at[:, pl.ds(qoff, B)], w_buf)
            copies = []
            for j in range(4):
                for kk in range(KSUB):
                    copies.append(pltpu.async_copy(
                        zrs_hbm.at[idx_buf.at[j, kk]],
                        g_buf.at[j, pl.ds(kk * CHUNK, CHUNK), :],
                        sem_g))
            for cp in copies:
                cp.wait()

            @pl.loop(0, B // L)
            def _vec(qi):
                q = qi * L + iota
                wv = [w_buf[j, pl.ds(qi * L, L)] for j in range(4)]
                for cc in range(8):
                    acc = None
                    for j in range(4):
                        g = plsc.load_gather(g_buf.at[j], [q, csplat[cc]])
                        acc = wv[j] * g if acc is None else acc + wv[j] * g
                    o_buf[cc, pl.ds(qi * L, L)] = acc

            pltpu.sync_copy(o_buf, out_hbm.at[:, pl.ds(qoff, B)])

    return k(zrs, idx_t, w_t)


def kernel(z, weights, index):
    c, hh, ww = z.shape
    n = index.shape[0]
    zrs = z.reshape(c, hh * ww).T
    idx_t = index.T.reshape(4, n // CHUNK, CHUNK)
    w_t = weights.T
    return _interp(zrs, idx_t, w_t, n=n, c=c)


# speedup vs baseline: 5.1102x; 1.5516x over previous
import functools

import jax
import jax.numpy as jnp
from jax import lax
from jax.experimental import pallas as pl
from jax.experimental.pallas import tpu as pltpu
from jax.experimental.pallas import tpu_sc as plsc

NC = 2
NS = 16
L = 16
NW = NC * NS

B = 1024
CHUNK = 128
KSUB = B // CHUNK


@functools.partial(jax.jit, static_argnames=("n", "c"))
def _interp(zrs, idx_t, w_t, *, n, c):
    per_w = n // NW
    nblk = per_w // B
    mesh = plsc.VectorSubcoreMesh(core_axis_name="c", subcore_axis_name="s")

    @functools.partial(
        pl.kernel,
        out_type=jax.ShapeDtypeStruct((n // CHUNK, c, CHUNK), jnp.float32),
        mesh=mesh,
        scratch_types=[
            pltpu.VMEM((4, KSUB, CHUNK), jnp.int32),
            pltpu.VMEM((4, B), jnp.float32),
            pltpu.VMEM((4, B, 8), jnp.float32),
            pltpu.VMEM((KSUB, 8, CHUNK), jnp.float32),
            pltpu.SemaphoreType.DMA,
        ],
        compiler_params=pltpu.CompilerParams(
            needs_layout_passes=False, use_tc_tiling_on_sc=False),
    )
    def k(zrs_hbm, idx_hbm, w_hbm, out_hbm, idx_buf, w_buf, g_buf, o_buf, sem_g):
        wid = lax.axis_index("s") * NC + lax.axis_index("c")
        base = wid * per_w
        iota = lax.iota(jnp.int32, L)
        csplat = [jnp.full((L,), cc, jnp.int32) for cc in range(8)]

        @pl.loop(0, nblk)
        def _blk(s):
            qoff = pl.multiple_of(base + s * B, B)
            coff = pl.multiple_of(qoff // CHUNK, KSUB)
            pltpu.sync_copy(idx_hbm.at[:, pl.ds(coff, KSUB), :], idx_buf)
            pltpu.sync_copy(w_hbm.at[:, pl.ds(qoff, B)], w_buf)
            copies = []
            for j in range(4):
                for kk in range(KSUB):
                    copies.append(pltpu.async_copy(
                        zrs_hbm.at[idx_buf.at[j, kk]],
                        g_buf.at[j, pl.ds(kk * CHUNK, CHUNK), :],
                        sem_g))
            for cp in copies:
                cp.wait()

            @pl.loop(0, KSUB)
            def _vec(kk):
                for t in range(CHUNK // L):
                    q0 = kk * CHUNK + t * L
                    q = q0 + iota
                    wv = [w_buf[j, pl.ds(q0, L)] for j in range(4)]
                    for cc in range(8):
                        acc = None
                        for j in range(4):
                            g = plsc.load_gather(g_buf.at[j], [q, csplat[cc]])
                            acc = wv[j] * g if acc is None else acc + wv[j] * g
                        o_buf[kk, cc, pl.ds(t * L, L)] = acc

            pltpu.sync_copy(o_buf, out_hbm.at[pl.ds(coff, KSUB)])

    out3 = k(zrs, idx_t, w_t)
    return out3.transpose(1, 0, 2).reshape(c, n)


def kernel(z, weights, index):
    c, hh, ww = z.shape
    n = index.shape[0]
    zrs = z.reshape(c, hh * ww).T
    idx_t = index.T.reshape(4, n // CHUNK, CHUNK)
    w_t = weights.T
    return _interp(zrs, idx_t, w_t, n=n, c=c)


# speedup vs baseline: 8.6193x; 1.6867x over previous
import functools

import jax
import jax.numpy as jnp
from jax import lax
from jax.experimental import pallas as pl
from jax.experimental.pallas import tpu as pltpu
from jax.experimental.pallas import tpu_sc as plsc

NC = 2
NS = 16
L = 16
NW = NC * NS

B = 1024
CHUNK = 128
KSUB = B // CHUNK


@functools.partial(jax.jit, static_argnames=("h", "w"))
def _build_table(zt5, *, h, w):
    yb_n = h // 8
    xb_n = w // 128
    units = yb_n * xb_n
    per_w = units // NW
    mesh = plsc.VectorSubcoreMesh(core_axis_name="c", subcore_axis_name="s")

    @functools.partial(
        pl.kernel,
        out_type=jax.ShapeDtypeStruct((h, xb_n, 1024), jnp.float32),
        mesh=mesh,
        scratch_types=[
            pltpu.VMEM((2, 8, 8, 128), jnp.float32),
            pltpu.VMEM((2, 8, 1024), jnp.float32),
            pltpu.SemaphoreType.DMA,
            pltpu.SemaphoreType.DMA,
            pltpu.SemaphoreType.DMA,
            pltpu.SemaphoreType.DMA,
        ],
        compiler_params=pltpu.CompilerParams(
            needs_layout_passes=False, use_tc_tiling_on_sc=False),
    )
    def k(zt_hbm, tab_hbm, in_buf, out_buf, sem_i0, sem_i1, sem_o0, sem_o1):
        wid = lax.axis_index("s") * NC + lax.axis_index("c")
        ubase = wid * per_w
        iota = lax.iota(jnp.int32, L)
        cvec = iota % 8
        xpair = iota // 8
        sem_i = [sem_i0, sem_i1]
        sem_o = [sem_o0, sem_o1]

        def unit_yx(u):
            uu = ubase + u
            return uu // xb_n, uu % xb_n

        def make_in(u, p):
            yb, xb = unit_yx(u)
            return pltpu.make_async_copy(
                zt_hbm.at[:, yb, xb], in_buf.at[p], sem_i[p])

        def make_out(u, p):
            yb, xb = unit_yx(u)
            return pltpu.make_async_copy(
                out_buf.at[p], tab_hbm.at[pl.ds(yb * 8, 8), xb], sem_o[p])

        def compute(u, p):
            @pl.loop(0, 8)
            def _yr(yr):
                yrs = jnp.zeros((L,), jnp.int32) + yr
                for xp in range(64):
                    g = plsc.load_gather(
                        in_buf.at[p], [cvec, yrs, xpair + 2 * xp])
                    out_buf[p, yr, pl.ds(xp * 2 * 8, L)] = g

        def body(cur, p, wait_out, start_next):
            make_in(cur, p).wait()
            if wait_out:
                make_out(cur - 2, p).wait()
            compute(cur, p)
            if start_next:
                make_in(cur + 2, p).start()
            make_out(cur, p).start()

        make_in(0, 0).start()
        make_in(1, 1).start()
        body(0, 0, False, True)
        body(1, 1, False, True)

        @pl.loop(2, per_w - 2, step=2)
        def _u(u):
            for par in range(2):
                body(u + par, par, True, True)

        body(per_w - 2, 0, True, False)
        body(per_w - 1, 1, True, False)
        make_out(per_w - 2, 0).wait()
        make_out(per_w - 1, 1).wait()

    return k(zt5)


@functools.partial(jax.jit, static_argnames=("n", "c"))
def _interp(zrs, idx_t, w_t, *, n, c):
    per_w = n // NW
    nblk = per_w // B
    mesh = plsc.VectorSubcoreMesh(core_axis_name="c", subcore_axis_name="s")

    @functools.partial(
        pl.kernel,
        out_type=jax.ShapeDtypeStruct((n // CHUNK, c, CHUNK), jnp.float32),
        mesh=mesh,
        scratch_types=[
            pltpu.VMEM((4, KSUB, CHUNK), jnp.int32),
            pltpu.VMEM((4, B), jnp.float32),
            pltpu.VMEM((4, B, 8), jnp.float32),
            pltpu.VMEM((KSUB, 8, CHUNK), jnp.float32),
            pltpu.SemaphoreType.DMA,
        ],
        compiler_params=pltpu.CompilerParams(
            needs_layout_passes=False, use_tc_tiling_on_sc=False),
    )
    def k(zrs_hbm, idx_hbm, w_hbm, out_hbm, idx_buf, w_buf, g_buf, o_buf, sem_g):
        wid = lax.axis_index("s") * NC + lax.axis_index("c")
        base = wid * per_w
        iota = lax.iota(jnp.int32, L)
        csplat = [jnp.full((L,), cc, jnp.int32) for cc in range(8)]

        @pl.loop(0, nblk)
        def _blk(s):
            qoff = pl.multiple_of(base + s * B, B)
            coff = pl.multiple_of(qoff // CHUNK, KSUB)
            pltpu.sync_copy(idx_hbm.at[:, pl.ds(coff, KSUB), :], idx_buf)
            pltpu.sync_copy(w_hbm.at[:, pl.ds(qoff, B)], w_buf)
            copies = []
            for j in range(4):
                for kk in range(KSUB):
                    copies.append(pltpu.async_copy(
                        zrs_hbm.at[idx_buf.at[j, kk]],
                        g_buf.at[j, pl.ds(kk * CHUNK, CHUNK), :],
                        sem_g))
            for cp in copies:
                cp.wait()

            @pl.loop(0, KSUB)
            def _vec(kk):
                for t in range(CHUNK // L):
                    q0 = kk * CHUNK + t * L
                    q = q0 + iota
                    wv = [w_buf[j, pl.ds(q0, L)] for j in range(4)]
                    for cc in range(8):
                        acc = None
                        for j in range(4):
                            g = plsc.load_gather(g_buf.at[j], [q, csplat[cc]])
                            acc = wv[j] * g if acc is None else acc + wv[j] * g
                        o_buf[kk, cc, pl.ds(t * L, L)] = acc

            pltpu.sync_copy(o_buf, out_hbm.at[pl.ds(coff, KSUB)])

    out3 = k(zrs, idx_t, w_t)
    return out3.transpose(1, 0, 2).reshape(c, n)


def kernel(z, weights, index):
    c, hh, ww = z.shape
    n = index.shape[0]
    zt5 = z.reshape(c, hh // 8, 8, ww // 128, 128).transpose(0, 1, 3, 2, 4)
    tab = _build_table(zt5, h=hh, w=ww)
    zrs = tab.reshape(hh * ww, c)
    idx_t = index.T.reshape(4, n // CHUNK, CHUNK)
    w_t = weights.T
    return _interp(zrs, idx_t, w_t, n=n, c=c)


# speedup vs baseline: 9.5354x; 1.1063x over previous
import functools

import jax
import jax.numpy as jnp
from jax import lax
from jax.experimental import pallas as pl
from jax.experimental.pallas import tpu as pltpu
from jax.experimental.pallas import tpu_sc as plsc

NC = 2
NS = 16
L = 16
NW = NC * NS

B = 1024
CHUNK = 128
KSUB = B // CHUNK


@functools.partial(jax.jit, static_argnames=("h", "w"))
def _build_table(zt5, *, h, w):
    yb_n = h // 8
    xb_n = w // 128
    units = yb_n * xb_n
    per_w = units // NW
    mesh = plsc.VectorSubcoreMesh(core_axis_name="c", subcore_axis_name="s")

    @functools.partial(
        pl.kernel,
        out_type=jax.ShapeDtypeStruct((h, xb_n, 1024), jnp.float32),
        mesh=mesh,
        scratch_types=[
            pltpu.VMEM((2, 8, 1033), jnp.float32),
            pltpu.VMEM((2, 8, 1024), jnp.float32),
            pltpu.SemaphoreType.DMA,
            pltpu.SemaphoreType.DMA,
            pltpu.SemaphoreType.DMA,
            pltpu.SemaphoreType.DMA,
        ],
        compiler_params=pltpu.CompilerParams(
            needs_layout_passes=False, use_tc_tiling_on_sc=False),
    )
    def k(zt_hbm, tab_hbm, in_buf, out_buf, sem_i0, sem_i1, sem_o0, sem_o1):
        wid = lax.axis_index("s") * NC + lax.axis_index("c")
        ubase = wid * per_w
        iota = lax.iota(jnp.int32, L)
        cvec = iota % 8
        xpair = iota // 8
        sem_i = [sem_i0, sem_i1]
        sem_o = [sem_o0, sem_o1]

        def unit_yx(u):
            uu = ubase + u
            return uu // xb_n, uu % xb_n

        def make_in(u, p):
            yb, xb = unit_yx(u)
            return pltpu.make_async_copy(
                zt_hbm.at[:, yb, xb], in_buf.at[p, :, pl.ds(0, 1024)], sem_i[p])

        def make_out(u, p):
            yb, xb = unit_yx(u)
            return pltpu.make_async_copy(
                out_buf.at[p], tab_hbm.at[pl.ds(yb * 8, 8), xb], sem_o[p])

        def compute(u, p):
            @pl.loop(0, 8)
            def _yr(yr):
                pos0 = yr * 128 + xpair
                for xp in range(64):
                    g = plsc.load_gather(
                        in_buf.at[p], [cvec, pos0 + 2 * xp])
                    out_buf[p, yr, pl.ds(xp * 2 * 8, L)] = g

        def body(cur, p, wait_out, start_next):
            make_in(cur, p).wait()
            if wait_out:
                make_out(cur - 2, p).wait()
            compute(cur, p)
            if start_next:
                make_in(cur + 2, p).start()
            make_out(cur, p).start()

        make_in(0, 0).start()
        make_in(1, 1).start()
        body(0, 0, False, True)
        body(1, 1, False, True)

        @pl.loop(2, per_w - 2, step=2)
        def _u(u):
            for par in range(2):
                body(u + par, par, True, True)

        body(per_w - 2, 0, True, False)
        body(per_w - 1, 1, True, False)
        make_out(per_w - 2, 0).wait()
        make_out(per_w - 1, 1).wait()

    return k(zt5)


@functools.partial(jax.jit, static_argnames=("n", "c"))
def _interp(zrs, idx_t, w_t, *, n, c):
    per_w = n // NW
    nblk = per_w // B
    mesh = plsc.VectorSubcoreMesh(core_axis_name="c", subcore_axis_name="s")

    @functools.partial(
        pl.kernel,
        out_type=jax.ShapeDtypeStruct((n // CHUNK, c, CHUNK), jnp.float32),
        mesh=mesh,
        scratch_types=[
            pltpu.VMEM((4, KSUB, CHUNK), jnp.int32),
            pltpu.VMEM((4, B), jnp.float32),
            pltpu.VMEM((4, B, 8), jnp.float32),
            pltpu.VMEM((KSUB, 8, CHUNK), jnp.float32),
            pltpu.SemaphoreType.DMA,
        ],
        compiler_params=pltpu.CompilerParams(
            needs_layout_passes=False, use_tc_tiling_on_sc=False),
    )
    def k(zrs_hbm, idx_hbm, w_hbm, out_hbm, idx_buf, w_buf, g_buf, o_buf, sem_g):
        wid = lax.axis_index("s") * NC + lax.axis_index("c")
        base = wid * per_w
        iota = lax.iota(jnp.int32, L)
        csplat = [jnp.full((L,), cc, jnp.int32) for cc in range(8)]

        @pl.loop(0, nblk)
        def _blk(s):
            qoff = pl.multiple_of(base + s * B, B)
            coff = pl.multiple_of(qoff // CHUNK, KSUB)
            pltpu.sync_copy(idx_hbm.at[:, pl.ds(coff, KSUB), :], idx_buf)
            pltpu.sync_copy(w_hbm.at[:, pl.ds(qoff, B)], w_buf)
            copies = []
            for j in range(4):
                for kk in range(KSUB):
                    copies.append(pltpu.async_copy(
                        zrs_hbm.at[idx_buf.at[j, kk]],
                        g_buf.at[j, pl.ds(kk * CHUNK, CHUNK), :],
                        sem_g))
            for cp in copies:
                cp.wait()

            @pl.loop(0, KSUB)
            def _vec(kk):
                for t in range(CHUNK // L):
                    q0 = kk * CHUNK + t * L
                    q = q0 + iota
                    wv = [w_buf[j, pl.ds(q0, L)] for j in range(4)]
                    for cc in range(8):
                        acc = None
                        for j in range(4):
                            g = plsc.load_gather(g_buf.at[j], [q, csplat[cc]])
                            acc = wv[j] * g if acc is None else acc + wv[j] * g
                        o_buf[kk, cc, pl.ds(t * L, L)] = acc

            pltpu.sync_copy(o_buf, out_hbm.at[pl.ds(coff, KSUB)])

    out3 = k(zrs, idx_t, w_t)
    return out3.transpose(1, 0, 2).reshape(c, n)


def kernel(z, weights, index):
    c, hh, ww = z.shape
    n = index.shape[0]
    zt5 = z.reshape(c, hh // 8, 8, ww // 128, 128).transpose(
        0, 1, 3, 2, 4).reshape(c, hh // 8, ww // 128, 1024)
    tab = _build_table(zt5, h=hh, w=ww)
    zrs = tab.reshape(hh * ww, c)
    idx_t = index.T.reshape(4, n // CHUNK, CHUNK)
    w_t = weights.T
    return _interp(zrs, idx_t, w_t, n=n, c=c)
